# baseline (speedup 1.0000x reference)
import functools

import jax
import jax.numpy as jnp
from jax import lax
from jax.experimental import pallas as pl
from jax.experimental.pallas import tpu as pltpu
from jax.experimental.pallas import tpu_sc as plsc

B, H, W, C = 4, 224, 224, 96
CP = 128
HW = H * W
N = B * HW
NW = 32
PPW = N // NW
K = 128
NCHUNK = PPW // K
G = K // 16
CG = C // 16


def _st_body(vol_hbm, ty_hbm, tx_hbm, out_hbm,
             ty_v, tx_v,
             i00, i01, i10, i11,
             w00, w01, w10, w11,
             r00, r01, r10, r11, ro,
             gsem):
  cid = lax.axis_index("c")
  sid = lax.axis_index("s")
  wid = sid * 2 + cid
  pix0 = wid * PPW
  vol_base = (pix0 // HW) * HW

  def chunk_body(g, carry):
    start = pix0 + g * K
    pltpu.sync_copy(ty_hbm.at[pl.ds(start, K)], ty_v)
    pltpu.sync_copy(tx_hbm.at[pl.ds(start, K)], tx_v)

    for j in range(G):
      sl = pl.ds(j * 16, 16)
      ty = ty_v[sl]
      tx = tx_v[sl]
      y0t = ty.astype(jnp.int32)
      y0t = jnp.where(y0t.astype(jnp.float32) > ty, y0t - 1, y0t)
      x0t = tx.astype(jnp.int32)
      x0t = jnp.where(x0t.astype(jnp.float32) > tx, x0t - 1, x0t)
      y0 = jnp.clip(y0t, 0, H - 1)
      y1 = jnp.clip(y0t + 1, 0, H - 1)
      x0 = jnp.clip(x0t, 0, W - 1)
      x1 = jnp.clip(x0t + 1, 0, W - 1)
      tyc = jnp.clip(ty, 0.0, float(H - 1))
      txc = jnp.clip(tx, 0.0, float(W - 1))
      wy0 = y1.astype(jnp.float32) - tyc
      wy1 = 1.0 - wy0
      wx0 = x1.astype(jnp.float32) - txc
      wx1 = 1.0 - wx0
      yb0 = vol_base + y0 * W
      yb1 = vol_base + y1 * W
      i00[sl] = yb0 + x0
      i01[sl] = yb0 + x1
      i10[sl] = yb1 + x0
      i11[sl] = yb1 + x1
      w00[sl] = wy0 * wx0
      w01[sl] = wy0 * wx1
      w10[sl] = wy1 * wx0
      w11[sl] = wy1 * wx1

    c0 = pltpu.async_copy(vol_hbm.at[i00], r00, gsem)
    c1 = pltpu.async_copy(vol_hbm.at[i01], r01, gsem)
    c2 = pltpu.async_copy(vol_hbm.at[i10], r10, gsem)
    c3 = pltpu.async_copy(vol_hbm.at[i11], r11, gsem)
    c0.wait()
    c1.wait()
    c2.wait()
    c3.wait()

    def grp_body(jj, c):
      sl = pl.ds(jj * 16, 16)
      wv00 = w00[sl]
      wv01 = w01[sl]
      wv10 = w10[sl]
      wv11 = w11[sl]
      base = jj * 16
      for i in range(16):
        p = base + i
        a00 = wv00[i]
        a01 = wv01[i]
        a10 = wv10[i]
        a11 = wv11[i]
        for cg in range(CG):
          s2 = pl.ds(cg * 16, 16)
          acc = a00 * r00[p, s2]
          acc = acc + a01 * r01[p, s2]
          acc = acc + a10 * r10[p, s2]
          acc = acc + a11 * r11[p, s2]
          ro[p, s2] = acc
      return c

    lax.fori_loop(0, G, grp_body, 0, unroll=False)

    pltpu.sync_copy(ro, out_hbm.at[pl.ds(start, K)])
    return carry

  lax.fori_loop(0, NCHUNK, chunk_body, 0, unroll=False)


@functools.partial(
    pl.kernel,
    mesh=plsc.VectorSubcoreMesh(core_axis_name="c", subcore_axis_name="s"),
    out_type=jax.ShapeDtypeStruct((N, C), jnp.float32),
    scratch_types=[
        pltpu.VMEM((K,), jnp.float32),
        pltpu.VMEM((K,), jnp.float32),
        pltpu.VMEM((K,), jnp.int32),
        pltpu.VMEM((K,), jnp.int32),
        pltpu.VMEM((K,), jnp.int32),
        pltpu.VMEM((K,), jnp.int32),
        pltpu.VMEM((K,), jnp.float32),
        pltpu.VMEM((K,), jnp.float32),
        pltpu.VMEM((K,), jnp.float32),
        pltpu.VMEM((K,), jnp.float32),
        pltpu.VMEM((K, CP), jnp.float32),
        pltpu.VMEM((K, CP), jnp.float32),
        pltpu.VMEM((K, CP), jnp.float32),
        pltpu.VMEM((K, CP), jnp.float32),
        pltpu.VMEM((K, C), jnp.float32),
        pltpu.SemaphoreType.DMA,
    ],
)
def _st_kernel(vol_hbm, ty_hbm, tx_hbm, out_hbm, *rest):
  _st_body(vol_hbm, ty_hbm, tx_hbm, out_hbm, *rest)


_BN = 2048


def _prep_body(vol_ref, trf_ref, volp_ref, ty_ref, tx_ref):
  volp_ref[:, :C] = vol_ref[...]
  t = trf_ref[...]
  ty_ref[...] = t[:, 0]
  tx_ref[...] = t[:, 1]


def _prep(vol2d, trf2d):
  return pl.pallas_call(
      _prep_body,
      grid=(N // _BN,),
      in_specs=[
          pl.BlockSpec((_BN, C), lambda i: (i, 0)),
          pl.BlockSpec((_BN, 2), lambda i: (i, 0)),
      ],
      out_specs=[
          pl.BlockSpec((_BN, CP), lambda i: (i, 0)),
          pl.BlockSpec((_BN,), lambda i: (i,)),
          pl.BlockSpec((_BN,), lambda i: (i,)),
      ],
      out_shape=[
          jax.ShapeDtypeStruct((N, CP), jnp.float32),
          jax.ShapeDtypeStruct((N,), jnp.float32),
          jax.ShapeDtypeStruct((N,), jnp.float32),
      ],
  )(vol2d, trf2d)


def kernel(vol, trf):
  vol_p, ty, tx = _prep(vol.reshape(N, C), trf.reshape(N, 2))
  out = _st_kernel(vol_p, ty, tx)
  return out.reshape(B, H, W, C)


# speedup vs baseline: 2.1532x; 2.1532x over previous
import functools

import jax
import jax.numpy as jnp
from jax import lax
from jax.experimental import pallas as pl
from jax.experimental.pallas import tpu as pltpu
from jax.experimental.pallas import tpu_sc as plsc

B, H, W, C = 4, 224, 224, 96
CP = 128
HW = H * W
N = B * HW
NW = 32
PPW = N // NW
K = 64
NCHUNK = PPW // K
NPAIR = NCHUNK // 2
G = K // 16
CG = C // 16


def _st_body(vol_hbm, ty_hbm, tx_hbm, out_hbm,
             ty_v, tx_v, dxv, dyv, wyv, wxv,
             i00a, i01a, i10a, i11a, w00a, w01a, w10a, w11a,
             i00b, i01b, i10b, i11b, w00b, w01b, w10b, w11b,
             r00a, r01a, r10a, r11a,
             r00b, r01b, r10b, r11b,
             ro, gsema, gsemb, osem):
  cid = lax.axis_index("c")
  sid = lax.axis_index("s")
  wid = sid * 2 + cid
  pix0 = wid * PPW
  vol_base = (pix0 // HW) * HW

  seta = (i00a, i01a, i10a, i11a, w00a, w01a, w10a, w11a,
          r00a, r01a, r10a, r11a, gsema)
  setb = (i00b, i01b, i10b, i11b, w00b, w01b, w10b, w11b,
          r00b, r01b, r10b, r11b, gsemb)

  def issue(g, bufs):
    (i00, i01, i10, i11, w00, w01, w10, w11,
     r00, r01, r10, r11, gsem) = bufs
    start = pix0 + g * K
    pltpu.sync_copy(ty_hbm.at[pl.ds(start, K)], ty_v)
    pltpu.sync_copy(tx_hbm.at[pl.ds(start, K)], tx_v)

    for j in range(G):
      sl = pl.ds(j * 16, 16)
      ty = ty_v[sl]
      tx = tx_v[sl]
      y0t = ty.astype(jnp.int32)
      y0t = jnp.where(y0t.astype(jnp.float32) > ty, y0t - 1, y0t)
      x0t = tx.astype(jnp.int32)
      x0t = jnp.where(x0t.astype(jnp.float32) > tx, x0t - 1, x0t)
      y0 = jnp.clip(y0t, 0, H - 1)
      y1 = jnp.clip(y0t + 1, 0, H - 1)
      x0 = jnp.clip(x0t, 0, W - 1)
      x1 = jnp.clip(x0t + 1, 0, W - 1)
      tyc = jnp.clip(ty, 0.0, float(H - 1))
      txc = jnp.clip(tx, 0.0, float(W - 1))
      i00[sl] = (vol_base + y0 * W) + x0
      dxv[sl] = x1 - x0
      dyv[sl] = (y1 - y0) * W
      wyv[sl] = y1.astype(jnp.float32) - tyc
      wxv[sl] = x1.astype(jnp.float32) - txc
    pltpu.async_copy(vol_hbm.at[i00], r00, gsem)

    for j in range(G):
      sl = pl.ds(j * 16, 16)
      i01[sl] = i00[sl] + dxv[sl]
    pltpu.async_copy(vol_hbm.at[i01], r01, gsem)
    for j in range(G):
      sl = pl.ds(j * 16, 16)
      i10[sl] = i00[sl] + dyv[sl]
    pltpu.async_copy(vol_hbm.at[i10], r10, gsem)
    for j in range(G):
      sl = pl.ds(j * 16, 16)
      i11[sl] = i10[sl] + dxv[sl]
    pltpu.async_copy(vol_hbm.at[i11], r11, gsem)

    for j in range(G):
      sl = pl.ds(j * 16, 16)
      wy0 = wyv[sl]
      wx0 = wxv[sl]
      m = wy0 * wx0
      w00[sl] = m
      w01[sl] = wy0 - m
      w10[sl] = wx0 - m
      w11[sl] = (1.0 - wy0) - (wx0 - m)

  def blend_and_write(g, bufs):
    (i00, i01, i10, i11, w00, w01, w10, w11,
     r00, r01, r10, r11, gsem) = bufs
    start = pix0 + g * K
    pltpu.make_async_copy(vol_hbm.at[i00], r00, gsem).wait()
    pltpu.make_async_copy(vol_hbm.at[i01], r01, gsem).wait()
    pltpu.make_async_copy(vol_hbm.at[i10], r10, gsem).wait()
    pltpu.make_async_copy(vol_hbm.at[i11], r11, gsem).wait()

    def grp_body(jj, c):
      sl = pl.ds(jj * 16, 16)
      wv00 = w00[sl]
      wv01 = w01[sl]
      wv10 = w10[sl]
      wv11 = w11[sl]
      base = jj * 16
      for i in range(16):
        p = base + i
        a00 = wv00[i]
        a01 = wv01[i]
        a10 = wv10[i]
        a11 = wv11[i]
        for cg in range(CG):
          s2 = pl.ds(cg * 16, 16)
          acc = a00 * r00[p, s2]
          acc = acc + a01 * r01[p, s2]
          acc = acc + a10 * r10[p, s2]
          acc = acc + a11 * r11[p, s2]
          ro[p, s2] = acc
      return c

    lax.fori_loop(0, G, grp_body, 0, unroll=False)
    pltpu.async_copy(ro, out_hbm.at[pl.ds(start, K)], osem)

  def wait_out(g):
    pltpu.make_async_copy(
        ro, out_hbm.at[pl.ds(pix0 + g * K, K)], osem).wait()

  issue(0, seta)

  def pair_body(p, carry):
    ga = 2 * p
    gb = ga + 1
    issue(gb, setb)

    @pl.when(p > 0)
    def _():
      wait_out(ga - 1)
    blend_and_write(ga, seta)

    @pl.when(p < NPAIR - 1)
    def _():
      issue(ga + 2, seta)

    wait_out(ga)
    blend_and_write(gb, setb)
    return carry

  lax.fori_loop(0, NPAIR, pair_body, 0, unroll=False)
  wait_out(NCHUNK - 1)


@functools.partial(
    pl.kernel,
    mesh=plsc.VectorSubcoreMesh(core_axis_name="c", subcore_axis_name="s"),
    out_type=jax.ShapeDtypeStruct((N, C), jnp.float32),
    scratch_types=[
        pltpu.VMEM((K,), jnp.float32),
        pltpu.VMEM((K,), jnp.float32),
        pltpu.VMEM((K,), jnp.int32),
        pltpu.VMEM((K,), jnp.int32),
        pltpu.VMEM((K,), jnp.float32),
        pltpu.VMEM((K,), jnp.float32),
        pltpu.VMEM((K,), jnp.int32),
        pltpu.VMEM((K,), jnp.int32),
        pltpu.VMEM((K,), jnp.int32),
        pltpu.VMEM((K,), jnp.int32),
        pltpu.VMEM((K,), jnp.float32),
        pltpu.VMEM((K,), jnp.float32),
        pltpu.VMEM((K,), jnp.float32),
        pltpu.VMEM((K,), jnp.float32),
        pltpu.VMEM((K,), jnp.int32),
        pltpu.VMEM((K,), jnp.int32),
        pltpu.VMEM((K,), jnp.int32),
        pltpu.VMEM((K,), jnp.int32),
        pltpu.VMEM((K,), jnp.float32),
        pltpu.VMEM((K,), jnp.float32),
        pltpu.VMEM((K,), jnp.float32),
        pltpu.VMEM((K,), jnp.float32),
        pltpu.VMEM((K, CP), jnp.float32),
        pltpu.VMEM((K, CP), jnp.float32),
        pltpu.VMEM((K, CP), jnp.float32),
        pltpu.VMEM((K, CP), jnp.float32),
        pltpu.VMEM((K, CP), jnp.float32),
        pltpu.VMEM((K, CP), jnp.float32),
        pltpu.VMEM((K, CP), jnp.float32),
        pltpu.VMEM((K, CP), jnp.float32),
        pltpu.VMEM((K, C), jnp.float32),
        pltpu.SemaphoreType.DMA,
        pltpu.SemaphoreType.DMA,
        pltpu.SemaphoreType.DMA,
    ],
)
def _st_kernel(vol_hbm, ty_hbm, tx_hbm, out_hbm, *rest):
  _st_body(vol_hbm, ty_hbm, tx_hbm, out_hbm, *rest)


_BH = 32
_BN = _BH * W


def _prep_body(vol_ref, trf_ref, volp_ref, ty_ref, tx_ref):
  v = vol_ref[0]
  volp_ref[:, :C] = jnp.swapaxes(v, 1, 2).reshape(_BN, C)
  t = trf_ref[0]
  for k in range(_BH):
    ty_ref[pl.ds(k * W, W)] = t[k, 0, :]
    tx_ref[pl.ds(k * W, W)] = t[k, 1, :]


def _prep(vol_t, trf_t):
  return pl.pallas_call(
      _prep_body,
      grid=(B, H // _BH),
      in_specs=[
          pl.BlockSpec((1, _BH, C, W), lambda b, h: (b, h, 0, 0)),
          pl.BlockSpec((1, _BH, 2, W), lambda b, h: (b, h, 0, 0)),
      ],
      out_specs=[
          pl.BlockSpec((_BN, CP), lambda b, h: (b * (H // _BH) + h, 0)),
          pl.BlockSpec((_BN,), lambda b, h: (b * (H // _BH) + h,)),
          pl.BlockSpec((_BN,), lambda b, h: (b * (H // _BH) + h,)),
      ],
      out_shape=[
          jax.ShapeDtypeStruct((N, CP), jnp.float32),
          jax.ShapeDtypeStruct((N,), jnp.float32),
          jax.ShapeDtypeStruct((N,), jnp.float32),
      ],
  )(vol_t, trf_t)


def _epilogue_body(out_ref, outt_ref):
  o = out_ref[...]
  outt_ref[0] = jnp.swapaxes(o.reshape(_BH, W, C), 1, 2)


def _epilogue(out):
  return pl.pallas_call(
      _epilogue_body,
      grid=(B, H // _BH),
      in_specs=[
          pl.BlockSpec((_BN, C), lambda b, h: (b * (H // _BH) + h, 0)),
      ],
      out_specs=pl.BlockSpec((1, _BH, C, W), lambda b, h: (b, h, 0, 0)),
      out_shape=jax.ShapeDtypeStruct((B, H, C, W), jnp.float32),
  )(out)


def kernel(vol, trf):
  vol_t = jnp.transpose(vol, (0, 1, 3, 2))
  trf_t = jnp.transpose(trf, (0, 1, 3, 2))
  vol_p, ty, tx = _prep(vol_t, trf_t)
  out = _st_kernel(vol_p, ty, tx)
  out_t = _epilogue(out)
  return jnp.transpose(out_t, (0, 1, 3, 2))
